# Initial kernel scaffold; baseline (speedup 1.0000x reference)
import jax
import jax.numpy as jnp
from jax.experimental import pallas as pl


def kernel(hidden_states, active_mask, W, b):
    raise NotImplementedError("write your pallas kernel here")


# speedup vs baseline: 1.8503x; 1.8503x over previous
import functools

import jax
import jax.numpy as jnp
from jax.experimental import pallas as pl
from jax.experimental.pallas import tpu as pltpu

_CS = 512


def _select_topk_mask(skey, am_i32, nch, cs):
    a = jnp.sum(am_i32)
    k = (a + 1) // 2

    n_pos = jnp.sum((skey >= 0).astype(jnp.int32))
    v0 = jnp.where(n_pos >= k, jnp.int32(0), jnp.int32(-(2 ** 31)))

    def vbody(i, v):
        bit = jnp.left_shift(jnp.int32(1), jnp.int32(30) - i)
        trial = v + bit
        c = jnp.sum((skey >= trial).astype(jnp.int32))
        return jnp.where(c >= k, trial, v)

    v = jax.lax.fori_loop(0, 31, vbody, v0)

    n_gt = jnp.sum((skey > v).astype(jnp.int32))
    t = k - n_gt
    eq = skey == v
    idx = (
        jax.lax.broadcasted_iota(jnp.int32, (nch, cs), 0) * cs
        + jax.lax.broadcasted_iota(jnp.int32, (nch, cs), 1)
    )

    def mbody(i, m):
        bit = jnp.left_shift(jnp.int32(1), jnp.int32(11) - i)
        trial = m + bit
        f = jnp.sum((eq & (idx < trial)).astype(jnp.int32))
        return jnp.where(f < t, trial, m)

    m = jax.lax.fori_loop(0, 12, mbody, jnp.int32(0))

    return ((skey > v) | (eq & (idx <= m))) & (k > 0)


def _router_kernel(am_ref, h_ref, w_ref, out_ref, scores_scr, *, nch, cs):
    j = pl.program_id(1)

    h = h_ref[0]
    w = w_ref[...]
    scores = jax.lax.dot_general(
        h.astype(jnp.bfloat16),
        w.reshape(-1, 1).astype(jnp.bfloat16),
        (((1,), (0,)), ((), ())),
        preferred_element_type=jnp.float32,
    )
    scores_scr[pl.ds(j, 1), :] = scores.reshape(1, cs)

    @pl.when(j == nch - 1)
    def _():
        am = am_ref[0] != 0
        s = jnp.where(am, scores_scr[...], -jnp.inf)
        bits = jax.lax.bitcast_convert_type(s, jnp.int32)
        skey = jnp.where(bits >= 0, bits, bits ^ jnp.int32(0x7FFFFFFF))
        mask = _select_topk_mask(skey, am.astype(jnp.int32), nch, cs)
        out_ref[0] = mask


def kernel(hidden_states, active_mask, W, b):
    B, S, D = hidden_states.shape
    cs = _CS
    nch = S // cs

    am3 = active_mask.reshape(B, nch, cs).astype(jnp.int32)
    w2 = W.reshape(1, D)

    out = pl.pallas_call(
        functools.partial(_router_kernel, nch=nch, cs=cs),
        grid=(B, nch),
        in_specs=[
            pl.BlockSpec((1, nch, cs), lambda i, j: (i, 0, 0)),
            pl.BlockSpec((1, cs, D), lambda i, j: (i, j, 0)),
            pl.BlockSpec((1, D), lambda i, j: (0, 0)),
        ],
        out_specs=pl.BlockSpec((1, nch, cs), lambda i, j: (i, 0, 0)),
        out_shape=jax.ShapeDtypeStruct((B, nch, cs), jnp.bool_),
        scratch_shapes=[pltpu.VMEM((nch, cs), jnp.float32)],
    )(am3, hidden_states, w2)

    return out.reshape(B, S)
